# baseline (device time: 223478 ns/iter reference)
import jax
import jax.numpy as jnp
from jax import lax
from jax.experimental import pallas as pl
from jax.experimental.pallas import tpu as pltpu

N_DEV = 16
B_PER = 2
B = N_DEV * B_PER
SQ = 128
D = 512
H_PER = 4
DH = 64


def _neighbor_barrier(left, right):
    barrier_sem = pltpu.get_barrier_semaphore()
    for nbr in (left, right):
        pl.semaphore_signal(
            barrier_sem, inc=1,
            device_id=(nbr,), device_id_type=pl.DeviceIdType.MESH,
        )
    pl.semaphore_wait(barrier_sem, 2)


def _ag_body(x_ref, out_ref, copy_sem, send_sems, recv_sems):
    me = lax.axis_index("i")
    left = lax.rem(me + N_DEV - 1, N_DEV)
    right = lax.rem(me + 1, N_DEV)

    _neighbor_barrier(left, right)

    local = pltpu.make_async_copy(x_ref, out_ref.at[me], copy_sem)
    local.start()
    local.wait()

    for h in range(N_DEV - 1):
        c = lax.rem(me - h + N_DEV, N_DEV)
        rdma = pltpu.make_async_remote_copy(
            src_ref=out_ref.at[c],
            dst_ref=out_ref.at[c],
            send_sem=send_sems.at[h],
            recv_sem=recv_sems.at[h],
            device_id=(right,),
            device_id_type=pl.DeviceIdType.MESH,
        )
        rdma.start()
        rdma.wait()


def _ring_all_gather(x_bf16):
    return pl.pallas_call(
        _ag_body,
        out_shape=jax.ShapeDtypeStruct((N_DEV, B_PER, SQ, D), jnp.bfloat16),
        in_specs=[pl.BlockSpec(memory_space=pltpu.VMEM)],
        out_specs=pl.BlockSpec(memory_space=pltpu.VMEM),
        scratch_shapes=[
            pltpu.SemaphoreType.DMA,
            pltpu.SemaphoreType.DMA((N_DEV - 1,)),
            pltpu.SemaphoreType.DMA((N_DEV - 1,)),
        ],
        compiler_params=pltpu.CompilerParams(collective_id=0),
    )(x_bf16)


def _rs_body(p_ref, out_ref, sbuf_ref, comm_ref, send_sem, recv_sems):
    me = lax.axis_index("i")
    left = lax.rem(me + N_DEV - 1, N_DEV)
    right = lax.rem(me + 1, N_DEV)

    _neighbor_barrier(left, right)

    for s in range(N_DEV - 1):
        c = lax.rem(me - 1 - s + 2 * N_DEV, N_DEV)
        if s == 0:
            sbuf_ref[...] = p_ref[c]
        else:
            sbuf_ref[...] = p_ref[c] + comm_ref[s - 1]
        rdma = pltpu.make_async_remote_copy(
            src_ref=sbuf_ref,
            dst_ref=comm_ref.at[s],
            send_sem=send_sem,
            recv_sem=recv_sems.at[s],
            device_id=(right,),
            device_id_type=pl.DeviceIdType.MESH,
        )
        rdma.start()
        rdma.wait()

    out_ref[...] = p_ref[me] + comm_ref[N_DEV - 2]


def _ring_reduce_scatter(partial):
    return pl.pallas_call(
        _rs_body,
        out_shape=jax.ShapeDtypeStruct((B_PER, SQ, D), jnp.float32),
        in_specs=[pl.BlockSpec(memory_space=pltpu.VMEM)],
        out_specs=pl.BlockSpec(memory_space=pltpu.VMEM),
        scratch_shapes=[
            pltpu.VMEM((B_PER, SQ, D), jnp.float32),
            pltpu.VMEM((N_DEV - 1, B_PER, SQ, D), jnp.float32),
            pltpu.SemaphoreType.DMA,
            pltpu.SemaphoreType.DMA((N_DEV - 1,)),
        ],
        compiler_params=pltpu.CompilerParams(collective_id=1),
    )(partial)


def kernel(x, Wq, Wk, Wv, Wo):
    bf = jnp.bfloat16

    xg = _ring_all_gather(x.astype(bf))
    X = xg.reshape(B * SQ, D)

    Q = jnp.dot(X, Wq.astype(bf)).reshape(B, SQ, H_PER, DH)
    K = jnp.dot(X, Wk.astype(bf)).reshape(B, SQ, H_PER, DH)
    V = jnp.dot(X, Wv.astype(bf)).reshape(B, SQ, H_PER, DH)

    inv = 1.0 / (10000.0 ** (jnp.arange(0, DH, 2, dtype=jnp.float32) / DH))
    pos = jnp.arange(SQ, dtype=jnp.float32)[:, None] * inv[None, :]
    cos = jnp.repeat(jnp.cos(pos), 2, axis=-1)
    sin = jnp.repeat(jnp.sin(pos), 2, axis=-1)

    def rot(t):
        tf = t.astype(jnp.float32)
        t2 = tf.reshape(B, SQ, H_PER, DH // 2, 2)
        t_r = jnp.stack([-t2[..., 1], t2[..., 0]], axis=-1)
        t_r = t_r.reshape(B, SQ, H_PER, DH)
        out = tf * cos[None, :, None, :] + t_r * sin[None, :, None, :]
        return out.astype(bf)

    Qr = rot(Q)
    Kr = rot(K)

    s = jnp.einsum(
        "bihd,bjhd->bhij", Qr, Kr, preferred_element_type=jnp.float32
    ) * 0.125
    w = jax.nn.softmax(s, axis=-1)
    ctx = jnp.einsum(
        "bhij,bjhd->bihd", w.astype(bf), V, preferred_element_type=jnp.float32
    )
    ctx = ctx.reshape(B * SQ, H_PER * DH).astype(bf)

    partial = jnp.dot(ctx, Wo.astype(bf), preferred_element_type=jnp.float32)
    partial = partial.reshape(N_DEV, B_PER, SQ, D)

    return _ring_reduce_scatter(partial)


# device time: 137418 ns/iter; 1.6263x vs baseline; 1.6263x over previous
import jax
import jax.numpy as jnp
from jax import lax
from jax.experimental import pallas as pl
from jax.experimental.pallas import tpu as pltpu

N_DEV = 16
B_PER = 2
B = N_DEV * B_PER
SQ = 128
D = 512
H_PER = 4
DH = 64


def _neighbor_barrier(left, right):
    barrier_sem = pltpu.get_barrier_semaphore()
    for nbr in (left, right):
        pl.semaphore_signal(
            barrier_sem, inc=1,
            device_id=(nbr,), device_id_type=pl.DeviceIdType.MESH,
        )
    pl.semaphore_wait(barrier_sem, 2)


R_HOPS = N_DEV // 2
L_HOPS = N_DEV - 1 - R_HOPS


def _ag_body(x_ref, out_ref, copy_sem, sr_sems, rr_sems, sl_sems, rl_sems):
    me = lax.axis_index("i")
    left = lax.rem(me + N_DEV - 1, N_DEV)
    right = lax.rem(me + 1, N_DEV)

    _neighbor_barrier(left, right)

    local = pltpu.make_async_copy(x_ref, out_ref.at[me], copy_sem)
    local.start()
    local.wait()

    for h in range(R_HOPS):
        cr = lax.rem(me - h + N_DEV, N_DEV)
        rdma_r = pltpu.make_async_remote_copy(
            src_ref=out_ref.at[cr],
            dst_ref=out_ref.at[cr],
            send_sem=sr_sems.at[h],
            recv_sem=rr_sems.at[h],
            device_id=(right,),
            device_id_type=pl.DeviceIdType.MESH,
        )
        rdma_r.start()
        if h < L_HOPS:
            cl = lax.rem(me + h, N_DEV)
            rdma_l = pltpu.make_async_remote_copy(
                src_ref=out_ref.at[cl],
                dst_ref=out_ref.at[cl],
                send_sem=sl_sems.at[h],
                recv_sem=rl_sems.at[h],
                device_id=(left,),
                device_id_type=pl.DeviceIdType.MESH,
            )
            rdma_l.start()
            rdma_l.wait()
        rdma_r.wait()


def _ring_all_gather(x_bf16):
    return pl.pallas_call(
        _ag_body,
        out_shape=jax.ShapeDtypeStruct((N_DEV, B_PER, SQ, D), jnp.bfloat16),
        in_specs=[pl.BlockSpec(memory_space=pltpu.VMEM)],
        out_specs=pl.BlockSpec(memory_space=pltpu.VMEM),
        scratch_shapes=[
            pltpu.SemaphoreType.DMA,
            pltpu.SemaphoreType.DMA((R_HOPS,)),
            pltpu.SemaphoreType.DMA((R_HOPS,)),
            pltpu.SemaphoreType.DMA((L_HOPS,)),
            pltpu.SemaphoreType.DMA((L_HOPS,)),
        ],
        compiler_params=pltpu.CompilerParams(collective_id=0),
    )(x_bf16)


def _rs_body(p_ref, out_ref, sbr_ref, sbl_ref, cr_ref, cl_ref,
             sr_sems, rr_sems, sl_sems, rl_sems):
    me = lax.axis_index("i")
    left = lax.rem(me + N_DEV - 1, N_DEV)
    right = lax.rem(me + 1, N_DEV)

    _neighbor_barrier(left, right)

    for s in range(R_HOPS):
        cr = lax.rem(me + R_HOPS - s, N_DEV)
        if s == 0:
            sbr_ref[...] = p_ref[cr].astype(jnp.bfloat16)
        else:
            sbr_ref[...] = (
                p_ref[cr] + cr_ref[s - 1].astype(jnp.float32)
            ).astype(jnp.bfloat16)
        rdma_r = pltpu.make_async_remote_copy(
            src_ref=sbr_ref,
            dst_ref=cr_ref.at[s],
            send_sem=sr_sems.at[s],
            recv_sem=rr_sems.at[s],
            device_id=(right,),
            device_id_type=pl.DeviceIdType.MESH,
        )
        rdma_r.start()
        if s < L_HOPS:
            cl = lax.rem(me - L_HOPS + s + N_DEV, N_DEV)
            if s == 0:
                sbl_ref[...] = p_ref[cl].astype(jnp.bfloat16)
            else:
                sbl_ref[...] = (
                    p_ref[cl] + cl_ref[s - 1].astype(jnp.float32)
                ).astype(jnp.bfloat16)
            rdma_l = pltpu.make_async_remote_copy(
                src_ref=sbl_ref,
                dst_ref=cl_ref.at[s],
                send_sem=sl_sems.at[s],
                recv_sem=rl_sems.at[s],
                device_id=(left,),
                device_id_type=pl.DeviceIdType.MESH,
            )
            rdma_l.start()
            rdma_l.wait()
        rdma_r.wait()

    out_ref[...] = (
        p_ref[me]
        + cr_ref[R_HOPS - 1].astype(jnp.float32)
        + cl_ref[L_HOPS - 1].astype(jnp.float32)
    )


def _ring_reduce_scatter(partial):
    return pl.pallas_call(
        _rs_body,
        out_shape=jax.ShapeDtypeStruct((B_PER, SQ, D), jnp.float32),
        in_specs=[pl.BlockSpec(memory_space=pltpu.VMEM)],
        out_specs=pl.BlockSpec(memory_space=pltpu.VMEM),
        scratch_shapes=[
            pltpu.VMEM((B_PER, SQ, D), jnp.bfloat16),
            pltpu.VMEM((B_PER, SQ, D), jnp.bfloat16),
            pltpu.VMEM((R_HOPS, B_PER, SQ, D), jnp.bfloat16),
            pltpu.VMEM((L_HOPS, B_PER, SQ, D), jnp.bfloat16),
            pltpu.SemaphoreType.DMA((R_HOPS,)),
            pltpu.SemaphoreType.DMA((R_HOPS,)),
            pltpu.SemaphoreType.DMA((L_HOPS,)),
            pltpu.SemaphoreType.DMA((L_HOPS,)),
        ],
        compiler_params=pltpu.CompilerParams(collective_id=1),
    )(partial)


def kernel(x, Wq, Wk, Wv, Wo):
    bf = jnp.bfloat16

    xg = _ring_all_gather(x.astype(bf))
    X = xg.reshape(B * SQ, D)

    Q = jnp.dot(X, Wq.astype(bf)).reshape(B, SQ, H_PER, DH)
    K = jnp.dot(X, Wk.astype(bf)).reshape(B, SQ, H_PER, DH)
    V = jnp.dot(X, Wv.astype(bf)).reshape(B, SQ, H_PER, DH)

    inv = 1.0 / (10000.0 ** (jnp.arange(0, DH, 2, dtype=jnp.float32) / DH))
    pos = jnp.arange(SQ, dtype=jnp.float32)[:, None] * inv[None, :]
    cos = jnp.repeat(jnp.cos(pos), 2, axis=-1)
    sin = jnp.repeat(jnp.sin(pos), 2, axis=-1)

    def rot(t):
        tf = t.astype(jnp.float32)
        t2 = tf.reshape(B, SQ, H_PER, DH // 2, 2)
        t_r = jnp.stack([-t2[..., 1], t2[..., 0]], axis=-1)
        t_r = t_r.reshape(B, SQ, H_PER, DH)
        out = tf * cos[None, :, None, :] + t_r * sin[None, :, None, :]
        return out.astype(bf)

    Qr = rot(Q)
    Kr = rot(K)

    s = jnp.einsum(
        "bihd,bjhd->bhij", Qr, Kr, preferred_element_type=jnp.float32
    ) * 0.125
    w = jax.nn.softmax(s, axis=-1)
    ctx = jnp.einsum(
        "bhij,bjhd->bihd", w.astype(bf), V, preferred_element_type=jnp.float32
    )
    ctx = ctx.reshape(B * SQ, H_PER * DH).astype(bf)

    partial = jnp.dot(ctx, Wo.astype(bf), preferred_element_type=jnp.float32)
    partial = partial.reshape(N_DEV, B_PER, SQ, D)

    return _ring_reduce_scatter(partial)


# device time: 108499 ns/iter; 2.0597x vs baseline; 1.2665x over previous
import jax
import jax.numpy as jnp
from jax import lax
from jax.experimental import pallas as pl
from jax.experimental.pallas import tpu as pltpu

N_DEV = 16
B_PER = 2
SQ = 128
D = 512
H_PER = 4
DH = 64
T = B_PER * SQ
HD = H_PER * DH

R_HOPS = N_DEV // 2
L_HOPS = N_DEV - 1 - R_HOPS

F32 = jnp.float32
BF16 = jnp.bfloat16


def _body(x_ref, wq_ref, wk_ref, wv_ref, wo_ref, cos_ref, sina_ref, sinb_ref,
          out_ref, xg_ref, p_ref,
          copy_sem, ag_sr, ag_rr, ag_sl, ag_rl,
          sbr_ref, sbl_ref, cr_ref, cl_ref,
          rs_sr, rs_rr, rs_sl, rs_rl):
    me = lax.axis_index("i")
    left = lax.rem(me + N_DEV - 1, N_DEV)
    right = lax.rem(me + 1, N_DEV)

    barrier_sem = pltpu.get_barrier_semaphore()
    for nbr in (left, right):
        pl.semaphore_signal(
            barrier_sem, inc=1,
            device_id=(nbr,), device_id_type=pl.DeviceIdType.MESH,
        )
    pl.semaphore_wait(barrier_sem, 2)

    wq = wq_ref[...]
    wk = wk_ref[...]
    wv = wv_ref[...]
    wo = wo_ref[...]
    cosm = cos_ref[...]
    sina = sina_ref[...]
    sinb = sinb_ref[...]

    def rope(t):
        tm1 = jnp.concatenate([t[:, 1:], t[:, :1]], axis=1)
        tp1 = jnp.concatenate([t[:, -1:], t[:, :-1]], axis=1)
        return (t * cosm + tm1 * sina + tp1 * sinb).astype(BF16)

    def compute_chunk(c):
        xc = xg_ref[c]
        q = rope(jnp.dot(xc, wq, preferred_element_type=F32))
        k = rope(jnp.dot(xc, wk, preferred_element_type=F32))
        v = jnp.dot(xc, wv, preferred_element_type=F32).astype(BF16)

        s_list = []
        for b in range(B_PER):
            for h in range(H_PER):
                qh = q[b * SQ:(b + 1) * SQ, h * DH:(h + 1) * DH]
                kh = k[b * SQ:(b + 1) * SQ, h * DH:(h + 1) * DH]
                s_list.append(lax.dot_general(
                    qh, kh, (((1,), (1,)), ((), ())),
                    preferred_element_type=F32,
                ))
        s = jnp.concatenate(s_list, axis=0) * 0.125
        m = jnp.max(s, axis=-1, keepdims=True)
        e = jnp.exp(s - m)
        w = (e / jnp.sum(e, axis=-1, keepdims=True)).astype(BF16)

        ctx_rows = []
        for b in range(B_PER):
            ctx_heads = []
            for h in range(H_PER):
                i = b * H_PER + h
                wbh = w[i * SQ:(i + 1) * SQ, :]
                vh = v[b * SQ:(b + 1) * SQ, h * DH:(h + 1) * DH]
                ctx_heads.append(jnp.dot(
                    wbh, vh, preferred_element_type=F32).astype(BF16))
            ctx_rows.append(jnp.concatenate(ctx_heads, axis=1))
        ctx = jnp.concatenate(ctx_rows, axis=0)
        p_ref[c] = jnp.dot(ctx, wo, preferred_element_type=F32)

    local = pltpu.make_async_copy(x_ref, xg_ref.at[me], copy_sem)
    local.start()
    local.wait()

    def ag_rdma(c, h, to_right):
        return pltpu.make_async_remote_copy(
            src_ref=xg_ref.at[c],
            dst_ref=xg_ref.at[c],
            send_sem=(ag_sr if to_right else ag_sl).at[h],
            recv_sem=(ag_rr if to_right else ag_rl).at[h],
            device_id=(right if to_right else left,),
            device_id_type=pl.DeviceIdType.MESH,
        )

    for h in range(R_HOPS):
        rdma_r = ag_rdma(lax.rem(me - h + N_DEV, N_DEV), h, True)
        rdma_r.start()
        rdma_l = None
        if h < L_HOPS:
            rdma_l = ag_rdma(lax.rem(me + h, N_DEV), h, False)
            rdma_l.start()
        if h == 0:
            compute_chunk(me)
        else:
            compute_chunk(lax.rem(me - h + N_DEV, N_DEV))
            compute_chunk(lax.rem(me + h, N_DEV))
        if rdma_l is not None:
            rdma_l.wait()
        rdma_r.wait()
    compute_chunk(lax.rem(me + R_HOPS, N_DEV))

    for s in range(R_HOPS):
        cr = lax.rem(me + R_HOPS - s, N_DEV)
        if s == 0:
            sbr_ref[...] = p_ref[cr].astype(BF16)
        else:
            sbr_ref[...] = (p_ref[cr] + cr_ref[s - 1].astype(F32)).astype(BF16)
        rdma_r = pltpu.make_async_remote_copy(
            src_ref=sbr_ref,
            dst_ref=cr_ref.at[s],
            send_sem=rs_sr.at[s],
            recv_sem=rs_rr.at[s],
            device_id=(right,),
            device_id_type=pl.DeviceIdType.MESH,
        )
        rdma_r.start()
        if s < L_HOPS:
            cl = lax.rem(me - L_HOPS + s + N_DEV, N_DEV)
            if s == 0:
                sbl_ref[...] = p_ref[cl].astype(BF16)
            else:
                sbl_ref[...] = (
                    p_ref[cl] + cl_ref[s - 1].astype(F32)).astype(BF16)
            rdma_l = pltpu.make_async_remote_copy(
                src_ref=sbl_ref,
                dst_ref=cl_ref.at[s],
                send_sem=rs_sl.at[s],
                recv_sem=rs_rl.at[s],
                device_id=(left,),
                device_id_type=pl.DeviceIdType.MESH,
            )
            rdma_l.start()
            rdma_l.wait()
        rdma_r.wait()

    out_ref[...] = (
        p_ref[me]
        + cr_ref[R_HOPS - 1].astype(F32)
        + cl_ref[L_HOPS - 1].astype(F32)
    )


def kernel(x, Wq, Wk, Wv, Wo):
    inv = 1.0 / (10000.0 ** (jnp.arange(0, DH, 2, dtype=F32) / DH))
    pos = jnp.arange(SQ, dtype=F32)[:, None] * inv[None, :]
    cos = jnp.repeat(jnp.cos(pos), 2, axis=-1)
    sin = jnp.repeat(jnp.sin(pos), 2, axis=-1)
    cosm = jnp.tile(cos, (B_PER, H_PER))
    sinm = jnp.tile(sin, (B_PER, H_PER))
    even = (jnp.arange(HD) % 2 == 0)[None, :]
    sina = jnp.where(even, -sinm, 0.0)
    sinb = jnp.where(even, 0.0, sinm)

    out = pl.pallas_call(
        _body,
        out_shape=jax.ShapeDtypeStruct((T, D), F32),
        in_specs=[pl.BlockSpec(memory_space=pltpu.VMEM)] * 8,
        out_specs=pl.BlockSpec(memory_space=pltpu.VMEM),
        scratch_shapes=[
            pltpu.VMEM((N_DEV, T, D), BF16),
            pltpu.VMEM((N_DEV, T, D), F32),
            pltpu.SemaphoreType.DMA,
            pltpu.SemaphoreType.DMA((R_HOPS,)),
            pltpu.SemaphoreType.DMA((R_HOPS,)),
            pltpu.SemaphoreType.DMA((L_HOPS,)),
            pltpu.SemaphoreType.DMA((L_HOPS,)),
            pltpu.VMEM((T, D), BF16),
            pltpu.VMEM((T, D), BF16),
            pltpu.VMEM((R_HOPS, T, D), BF16),
            pltpu.VMEM((L_HOPS, T, D), BF16),
            pltpu.SemaphoreType.DMA((R_HOPS,)),
            pltpu.SemaphoreType.DMA((R_HOPS,)),
            pltpu.SemaphoreType.DMA((L_HOPS,)),
            pltpu.SemaphoreType.DMA((L_HOPS,)),
        ],
        compiler_params=pltpu.CompilerParams(collective_id=0),
    )(
        x.reshape(T, D).astype(BF16),
        Wq.astype(BF16), Wk.astype(BF16), Wv.astype(BF16), Wo.astype(BF16),
        cosm, sina, sinb,
    )
    return out.reshape(B_PER, SQ, D)


# device time: 86784 ns/iter; 2.5751x vs baseline; 1.2502x over previous
import jax
import jax.numpy as jnp
from jax import lax
from jax.experimental import pallas as pl
from jax.experimental.pallas import tpu as pltpu

N_DEV = 16
B_PER = 2
SQ = 128
D = 512
H_PER = 4
DH = 64
T = B_PER * SQ
HD = H_PER * DH

R_HOPS = N_DEV // 2
L_HOPS = N_DEV - 1 - R_HOPS

F32 = jnp.float32
BF16 = jnp.bfloat16


def _body(x_ref, wq_ref, wk_ref, wv_ref, wo_ref, cos_ref, sina_ref, sinb_ref,
          out_ref, xg_ref, p_ref,
          ag_sr, ag_rr, ag_sl, ag_rl,
          sbr_ref, sbl_ref, cr_ref, cl_ref,
          rs_sr, rs_rr, rs_sl, rs_rl):
    me = lax.axis_index("i")
    left = lax.rem(me + N_DEV - 1, N_DEV)
    right = lax.rem(me + 1, N_DEV)

    barrier_sem = pltpu.get_barrier_semaphore()
    for nbr in (left, right):
        pl.semaphore_signal(
            barrier_sem, inc=1,
            device_id=(nbr,), device_id_type=pl.DeviceIdType.MESH,
        )
    pl.semaphore_wait(barrier_sem, 2)

    wq = wq_ref[...]
    wk = wk_ref[...]
    wv = wv_ref[...]
    wo = wo_ref[...]
    cosm = cos_ref[...]
    sina = sina_ref[...]
    sinb = sinb_ref[...]

    def rope(t):
        tm1 = jnp.concatenate([t[:, 1:], t[:, :1]], axis=1)
        tp1 = jnp.concatenate([t[:, -1:], t[:, :-1]], axis=1)
        return (t * cosm + tm1 * sina + tp1 * sinb).astype(BF16)

    def compute_chunk(c, xc):
        q = rope(jnp.dot(xc, wq, preferred_element_type=F32))
        k = rope(jnp.dot(xc, wk, preferred_element_type=F32))
        v = jnp.dot(xc, wv, preferred_element_type=F32).astype(BF16)

        s_list = []
        for b in range(B_PER):
            for h in range(H_PER):
                qh = q[b * SQ:(b + 1) * SQ, h * DH:(h + 1) * DH]
                kh = k[b * SQ:(b + 1) * SQ, h * DH:(h + 1) * DH]
                s_list.append(lax.dot_general(
                    qh, kh, (((1,), (1,)), ((), ())),
                    preferred_element_type=F32,
                ))
        s = jnp.concatenate(s_list, axis=0) * 0.125
        m = jnp.max(s, axis=-1, keepdims=True)
        e = jnp.exp(s - m)
        w = (e / jnp.sum(e, axis=-1, keepdims=True)).astype(BF16)

        ctx_rows = []
        for b in range(B_PER):
            ctx_heads = []
            for h in range(H_PER):
                i = b * H_PER + h
                wbh = w[i * SQ:(i + 1) * SQ, :]
                vh = v[b * SQ:(b + 1) * SQ, h * DH:(h + 1) * DH]
                ctx_heads.append(jnp.dot(
                    wbh, vh, preferred_element_type=F32).astype(BF16))
            ctx_rows.append(jnp.concatenate(ctx_heads, axis=1))
        ctx = jnp.concatenate(ctx_rows, axis=0)
        partial = jnp.dot(ctx, wo, preferred_element_type=F32)
        p_ref[c * 2] = partial[:SQ, :]
        p_ref[c * 2 + 1] = partial[SQ:, :]

    def gathered(c):
        return jnp.concatenate([xg_ref[c * 2], xg_ref[c * 2 + 1]], axis=0)

    def ag_desc(c, h, j, to_right):
        src = x_ref.at[j] if h == 0 else xg_ref.at[c * 2 + j]
        return pltpu.make_async_remote_copy(
            src_ref=src,
            dst_ref=xg_ref.at[c * 2 + j],
            send_sem=(ag_sr if to_right else ag_sl).at[h * 2 + j],
            recv_sem=(ag_rr if to_right else ag_rl).at[h * 2 + j],
            device_id=(right if to_right else left,),
            device_id_type=pl.DeviceIdType.MESH,
        )

    r_descs = [[None, None] for _ in range(R_HOPS)]
    l_descs = [[None, None] for _ in range(L_HOPS)]
    for j in range(2):
        r_descs[0][j] = ag_desc(me, 0, j, True)
        r_descs[0][j].start()
        l_descs[0][j] = ag_desc(me, 0, j, False)
        l_descs[0][j].start()
    compute_chunk(me, x_ref[...].reshape(T, D))

    for h in range(1, R_HOPS):
        c_r = lax.rem(me - h + N_DEV, N_DEV)
        c_l = lax.rem(me + h, N_DEV)
        for j in range(2):
            r_descs[h - 1][j].wait_recv()
            r_descs[h][j] = ag_desc(c_r, h, j, True)
            r_descs[h][j].start()
        if h < L_HOPS:
            for j in range(2):
                l_descs[h - 1][j].wait_recv()
                l_descs[h][j] = ag_desc(c_l, h, j, False)
                l_descs[h][j].start()
        else:
            for j in range(2):
                l_descs[L_HOPS - 1][j].wait_recv()
        compute_chunk(c_r, gathered(c_r))
        compute_chunk(c_l, gathered(c_l))

    for j in range(2):
        r_descs[R_HOPS - 1][j].wait_recv()
    c_far = lax.rem(me + R_HOPS, N_DEV)
    compute_chunk(c_far, gathered(c_far))

    for descs in (r_descs, l_descs):
        for pair in descs:
            for d in pair:
                d.wait_send()

    rsr = [[None, None] for _ in range(R_HOPS)]
    rsl = [[None, None] for _ in range(L_HOPS)]
    for s in range(R_HOPS):
        c_r = lax.rem(me + R_HOPS - s, N_DEV)
        for j in range(2):
            if s >= 2:
                rsr[s - 2][j].wait_send()
            if s == 0:
                sbr_ref[(s % 2) * 2 + j] = p_ref[c_r * 2 + j].astype(BF16)
            else:
                rsr[s - 1][j].wait_recv()
                sbr_ref[(s % 2) * 2 + j] = (
                    p_ref[c_r * 2 + j]
                    + cr_ref[(s - 1) * 2 + j].astype(F32)
                ).astype(BF16)
            d = pltpu.make_async_remote_copy(
                src_ref=sbr_ref.at[(s % 2) * 2 + j],
                dst_ref=cr_ref.at[s * 2 + j],
                send_sem=rs_sr.at[s * 2 + j],
                recv_sem=rs_rr.at[s * 2 + j],
                device_id=(right,),
                device_id_type=pl.DeviceIdType.MESH,
            )
            d.start()
            rsr[s][j] = d
        if s < L_HOPS:
            c_l = lax.rem(me - L_HOPS + s + N_DEV, N_DEV)
            for j in range(2):
                if s >= 2:
                    rsl[s - 2][j].wait_send()
                if s == 0:
                    sbl_ref[(s % 2) * 2 + j] = p_ref[c_l * 2 + j].astype(BF16)
                else:
                    rsl[s - 1][j].wait_recv()
                    sbl_ref[(s % 2) * 2 + j] = (
                        p_ref[c_l * 2 + j]
                        + cl_ref[(s - 1) * 2 + j].astype(F32)
                    ).astype(BF16)
                d = pltpu.make_async_remote_copy(
                    src_ref=sbl_ref.at[(s % 2) * 2 + j],
                    dst_ref=cl_ref.at[s * 2 + j],
                    send_sem=rs_sl.at[s * 2 + j],
                    recv_sem=rs_rl.at[s * 2 + j],
                    device_id=(left,),
                    device_id_type=pl.DeviceIdType.MESH,
                )
                d.start()
                rsl[s][j] = d

    for j in range(2):
        rsr[R_HOPS - 1][j].wait_recv()
        rsl[L_HOPS - 1][j].wait_recv()
    for j in range(2):
        out_ref[j] = (
            p_ref[me * 2 + j]
            + cr_ref[(R_HOPS - 1) * 2 + j].astype(F32)
            + cl_ref[(L_HOPS - 1) * 2 + j].astype(F32)
        )
    for s in (R_HOPS - 2, R_HOPS - 1):
        for j in range(2):
            rsr[s][j].wait_send()
    for s in (L_HOPS - 2, L_HOPS - 1):
        for j in range(2):
            rsl[s][j].wait_send()


def kernel(x, Wq, Wk, Wv, Wo):
    inv = 1.0 / (10000.0 ** (jnp.arange(0, DH, 2, dtype=F32) / DH))
    pos = jnp.arange(SQ, dtype=F32)[:, None] * inv[None, :]
    cos = jnp.repeat(jnp.cos(pos), 2, axis=-1)
    sin = jnp.repeat(jnp.sin(pos), 2, axis=-1)
    cosm = jnp.tile(cos, (B_PER, H_PER))
    sinm = jnp.tile(sin, (B_PER, H_PER))
    even = (jnp.arange(HD) % 2 == 0)[None, :]
    sina = jnp.where(even, -sinm, 0.0)
    sinb = jnp.where(even, 0.0, sinm)

    return pl.pallas_call(
        _body,
        out_shape=jax.ShapeDtypeStruct((B_PER, SQ, D), F32),
        in_specs=[pl.BlockSpec(memory_space=pltpu.VMEM)] * 8,
        out_specs=pl.BlockSpec(memory_space=pltpu.VMEM),
        scratch_shapes=[
            pltpu.VMEM((N_DEV * 2, SQ, D), BF16),
            pltpu.VMEM((N_DEV * 2, SQ, D), F32),
            pltpu.SemaphoreType.DMA((R_HOPS * 2,)),
            pltpu.SemaphoreType.DMA((R_HOPS * 2,)),
            pltpu.SemaphoreType.DMA((L_HOPS * 2,)),
            pltpu.SemaphoreType.DMA((L_HOPS * 2,)),
            pltpu.VMEM((4, SQ, D), BF16),
            pltpu.VMEM((4, SQ, D), BF16),
            pltpu.VMEM((R_HOPS * 2, SQ, D), BF16),
            pltpu.VMEM((L_HOPS * 2, SQ, D), BF16),
            pltpu.SemaphoreType.DMA((R_HOPS * 2,)),
            pltpu.SemaphoreType.DMA((R_HOPS * 2,)),
            pltpu.SemaphoreType.DMA((L_HOPS * 2,)),
            pltpu.SemaphoreType.DMA((L_HOPS * 2,)),
        ],
        compiler_params=pltpu.CompilerParams(collective_id=0),
    )(
        x.astype(BF16),
        Wq.astype(BF16), Wk.astype(BF16), Wv.astype(BF16), Wo.astype(BF16),
        cosm, sina, sinb,
    )


# device time: 83503 ns/iter; 2.6763x vs baseline; 1.0393x over previous
import jax
import jax.numpy as jnp
from jax import lax
from jax.experimental import pallas as pl
from jax.experimental.pallas import tpu as pltpu

N_DEV = 16
B_PER = 2
SQ = 128
D = 512
H_PER = 4
DH = 64
T = B_PER * SQ
HD = H_PER * DH

R_HOPS = N_DEV // 2
L_HOPS = N_DEV - 1 - R_HOPS

F32 = jnp.float32
BF16 = jnp.bfloat16


def _body(x_ref, wq_ref, wk_ref, wv_ref, wo_ref, cos_ref, sina_ref, sinb_ref,
          out_ref, xg_ref, p_ref,
          ag_sr, ag_rr, ag_sl, ag_rl,
          sbr_ref, sbl_ref, cr_ref, cl_ref,
          rs_sr, rs_rr, rs_sl, rs_rl):
    me = lax.axis_index("i")
    left = lax.rem(me + N_DEV - 1, N_DEV)
    right = lax.rem(me + 1, N_DEV)

    barrier_sem = pltpu.get_barrier_semaphore()
    for nbr in (left, right):
        pl.semaphore_signal(
            barrier_sem, inc=1,
            device_id=(nbr,), device_id_type=pl.DeviceIdType.MESH,
        )
    pl.semaphore_wait(barrier_sem, 2)

    wq = wq_ref[...]
    wk = wk_ref[...]
    wv = wv_ref[...]
    wo = wo_ref[...]
    cosm = cos_ref[...]
    sina = sina_ref[...]
    sinb = sinb_ref[...]

    def rope(t):
        tm1 = jnp.concatenate([t[:, 1:], t[:, :1]], axis=1)
        tp1 = jnp.concatenate([t[:, -1:], t[:, :-1]], axis=1)
        return (t * cosm + tm1 * sina + tp1 * sinb).astype(BF16)

    def compute_chunk(c, xc):
        q = rope(jnp.dot(xc, wq, preferred_element_type=F32))
        k = rope(jnp.dot(xc, wk, preferred_element_type=F32))
        v = jnp.dot(xc, wv, preferred_element_type=F32).astype(BF16)

        s_list = []
        for b in range(B_PER):
            for h in range(H_PER):
                qh = q[b * SQ:(b + 1) * SQ, h * DH:(h + 1) * DH]
                kh = k[b * SQ:(b + 1) * SQ, h * DH:(h + 1) * DH]
                s_list.append(lax.dot_general(
                    qh, kh, (((1,), (1,)), ((), ())),
                    preferred_element_type=F32,
                ))
        s = jnp.concatenate(s_list, axis=0) * 0.125
        e = jnp.exp(s)
        denom = jnp.sum(e, axis=-1, keepdims=True)
        eb = e.astype(BF16)

        ctx_rows = []
        for b in range(B_PER):
            ctx_heads = []
            for h in range(H_PER):
                i = b * H_PER + h
                ebh = eb[i * SQ:(i + 1) * SQ, :]
                vh = v[b * SQ:(b + 1) * SQ, h * DH:(h + 1) * DH]
                raw = jnp.dot(ebh, vh, preferred_element_type=F32)
                ctx_heads.append(
                    (raw / denom[i * SQ:(i + 1) * SQ, :]).astype(BF16))
            ctx_rows.append(jnp.concatenate(ctx_heads, axis=1))
        ctx = jnp.concatenate(ctx_rows, axis=0)
        partial = jnp.dot(ctx, wo, preferred_element_type=F32)
        p_ref[c * 2] = partial[:SQ, :]
        p_ref[c * 2 + 1] = partial[SQ:, :]

    def gathered(c):
        return jnp.concatenate([xg_ref[c * 2], xg_ref[c * 2 + 1]], axis=0)

    def ag_desc(c, h, j, to_right):
        src = x_ref.at[j] if h == 0 else xg_ref.at[c * 2 + j]
        return pltpu.make_async_remote_copy(
            src_ref=src,
            dst_ref=xg_ref.at[c * 2 + j],
            send_sem=(ag_sr if to_right else ag_sl).at[h * 2 + j],
            recv_sem=(ag_rr if to_right else ag_rl).at[h * 2 + j],
            device_id=(right if to_right else left,),
            device_id_type=pl.DeviceIdType.MESH,
        )

    r_descs = [[None, None] for _ in range(R_HOPS)]
    l_descs = [[None, None] for _ in range(L_HOPS)]
    for j in range(2):
        r_descs[0][j] = ag_desc(me, 0, j, True)
        r_descs[0][j].start()
        l_descs[0][j] = ag_desc(me, 0, j, False)
        l_descs[0][j].start()

    for h in range(1, R_HOPS):
        c_r = lax.rem(me - h + N_DEV, N_DEV)
        c_l = lax.rem(me + h, N_DEV)
        for j in range(2):
            r_descs[h - 1][j].wait_recv()
            r_descs[h][j] = ag_desc(c_r, h, j, True)
            r_descs[h][j].start()
        if h < L_HOPS:
            for j in range(2):
                l_descs[h - 1][j].wait_recv()
                l_descs[h][j] = ag_desc(c_l, h, j, False)
                l_descs[h][j].start()
        else:
            for j in range(2):
                l_descs[L_HOPS - 1][j].wait_recv()
        compute_chunk(c_r, gathered(c_r))

    for j in range(2):
        r_descs[R_HOPS - 1][j].wait_recv()

    for descs in (r_descs, l_descs):
        for pair in descs:
            for d in pair:
                d.wait_send()

    rsr = [[None, None] for _ in range(R_HOPS)]
    rsl = [[None, None] for _ in range(L_HOPS)]

    def rs_send(s, j, c, to_right):
        sb_ref = sbr_ref if to_right else sbl_ref
        cm_ref = cr_ref if to_right else cl_ref
        if s == 0:
            sb_ref[(s % 2) * 2 + j] = p_ref[c * 2 + j].astype(BF16)
        else:
            (rsr if to_right else rsl)[s - 1][j].wait_recv()
            sb_ref[(s % 2) * 2 + j] = (
                p_ref[c * 2 + j]
                + cm_ref[(s - 1) * 2 + j].astype(F32)
            ).astype(BF16)
        d = pltpu.make_async_remote_copy(
            src_ref=sb_ref.at[(s % 2) * 2 + j],
            dst_ref=cm_ref.at[s * 2 + j],
            send_sem=(rs_sr if to_right else rs_sl).at[s * 2 + j],
            recv_sem=(rs_rr if to_right else rs_rl).at[s * 2 + j],
            device_id=(right if to_right else left,),
            device_id_type=pl.DeviceIdType.MESH,
        )
        d.start()
        (rsr if to_right else rsl)[s][j] = d

    c_l0 = lax.rem(me - L_HOPS + N_DEV, N_DEV)
    for j in range(2):
        rs_send(0, j, c_l0, False)
    c_far = lax.rem(me + R_HOPS, N_DEV)
    compute_chunk(c_far, gathered(c_far))
    for j in range(2):
        rs_send(0, j, c_far, True)

    for s in range(1, R_HOPS):
        c_r = lax.rem(me + R_HOPS - s, N_DEV)
        compute_chunk(c_r, gathered(c_r))
        for j in range(2):
            if s >= 2:
                rsr[s - 2][j].wait_send()
            rs_send(s, j, c_r, True)
        if s < L_HOPS:
            c_l = lax.rem(me - L_HOPS + s + N_DEV, N_DEV)
            for j in range(2):
                if s >= 2:
                    rsl[s - 2][j].wait_send()
                rs_send(s, j, c_l, False)

    compute_chunk(me, x_ref[...].reshape(T, D))
    for j in range(2):
        rsr[R_HOPS - 1][j].wait_recv()
        rsl[L_HOPS - 1][j].wait_recv()
    for j in range(2):
        out_ref[j] = (
            p_ref[me * 2 + j]
            + cr_ref[(R_HOPS - 1) * 2 + j].astype(F32)
            + cl_ref[(L_HOPS - 1) * 2 + j].astype(F32)
        )
    for s in (R_HOPS - 2, R_HOPS - 1):
        for j in range(2):
            rsr[s][j].wait_send()
    for s in (L_HOPS - 2, L_HOPS - 1):
        for j in range(2):
            rsl[s][j].wait_send()


def kernel(x, Wq, Wk, Wv, Wo):
    inv = 1.0 / (10000.0 ** (jnp.arange(0, DH, 2, dtype=F32) / DH))
    pos = jnp.arange(SQ, dtype=F32)[:, None] * inv[None, :]
    cos = jnp.repeat(jnp.cos(pos), 2, axis=-1)
    sin = jnp.repeat(jnp.sin(pos), 2, axis=-1)
    cosm = jnp.tile(cos, (B_PER, H_PER))
    sinm = jnp.tile(sin, (B_PER, H_PER))
    even = (jnp.arange(HD) % 2 == 0)[None, :]
    sina = jnp.where(even, -sinm, 0.0)
    sinb = jnp.where(even, 0.0, sinm)

    return pl.pallas_call(
        _body,
        out_shape=jax.ShapeDtypeStruct((B_PER, SQ, D), F32),
        in_specs=[pl.BlockSpec(memory_space=pltpu.VMEM)] * 8,
        out_specs=pl.BlockSpec(memory_space=pltpu.VMEM),
        scratch_shapes=[
            pltpu.VMEM((N_DEV * 2, SQ, D), BF16),
            pltpu.VMEM((N_DEV * 2, SQ, D), F32),
            pltpu.SemaphoreType.DMA((R_HOPS * 2,)),
            pltpu.SemaphoreType.DMA((R_HOPS * 2,)),
            pltpu.SemaphoreType.DMA((L_HOPS * 2,)),
            pltpu.SemaphoreType.DMA((L_HOPS * 2,)),
            pltpu.VMEM((4, SQ, D), BF16),
            pltpu.VMEM((4, SQ, D), BF16),
            pltpu.VMEM((R_HOPS * 2, SQ, D), BF16),
            pltpu.VMEM((L_HOPS * 2, SQ, D), BF16),
            pltpu.SemaphoreType.DMA((R_HOPS * 2,)),
            pltpu.SemaphoreType.DMA((R_HOPS * 2,)),
            pltpu.SemaphoreType.DMA((L_HOPS * 2,)),
            pltpu.SemaphoreType.DMA((L_HOPS * 2,)),
        ],
        compiler_params=pltpu.CompilerParams(collective_id=0),
    )(
        x.astype(BF16),
        Wq.astype(BF16), Wk.astype(BF16), Wv.astype(BF16), Wo.astype(BF16),
        cosm, sina, sinb,
    )
